# Initial kernel scaffold; baseline (speedup 1.0000x reference)
import jax
import jax.numpy as jnp
from jax.experimental import pallas as pl


def kernel(x, edge_index, edge_attr, embed_table, W1, b1, Wh, bh):
    raise NotImplementedError("write your pallas kernel here")


# speedup vs baseline: 3.1563x; 3.1563x over previous
import functools

import jax
import jax.numpy as jnp
from jax import lax
from jax.experimental import pallas as pl
from jax.experimental.pallas import tpu as pltpu
from jax.experimental.pallas import tpu_sc as plsc

N_NODES = 10000
D_FEAT = 128
EMB = 16
EDGE_DIM = 16
HIDDEN = 64
NUM_EMB = 8

NW = 32
CHUNK = 128
CPW = 80
NCH = NW * CPW
EP = NCH * CHUNK
STRIPE = 632
ACC_ROWS = 16 * STRIPE

_SC_MESH = plsc.VectorSubcoreMesh(core_axis_name="c", subcore_axis_name="s")


def _node_body(xa_ref, t_ref, emb_ref, w1a_ref, w1b_ref, b1_ref, u_ref):
    emb_w = jnp.dot(emb_ref[...], w1b_ref[...],
                    preferred_element_type=jnp.float32)
    t = t_ref[...].astype(jnp.int32)
    onehot = (lax.broadcasted_iota(jnp.int32, (t.shape[0], NUM_EMB), 1)
              == t).astype(jnp.float32)
    u = (jnp.dot(xa_ref[...], w1a_ref[...],
                 preferred_element_type=jnp.float32)
         + jnp.dot(onehot, emb_w, preferred_element_type=jnp.float32)
         + b1_ref[...])
    u_ref[...] = u


def _node_precompute(xa, t, embed_table, w1a, w1b, b1):
    bn = 1000
    grid = N_NODES // bn
    return pl.pallas_call(
        _node_body,
        grid=(grid,),
        in_specs=[
            pl.BlockSpec((bn, D_FEAT), lambda i: (i, 0)),
            pl.BlockSpec((bn, 1), lambda i: (i, 0)),
            pl.BlockSpec((NUM_EMB, EMB), lambda i: (0, 0)),
            pl.BlockSpec((D_FEAT, HIDDEN), lambda i: (0, 0)),
            pl.BlockSpec((EMB, HIDDEN), lambda i: (0, 0)),
            pl.BlockSpec((1, HIDDEN), lambda i: (0, 0)),
        ],
        out_specs=pl.BlockSpec((bn, HIDDEN), lambda i: (i, 0)),
        out_shape=jax.ShapeDtypeStruct((N_NODES, HIDDEN), jnp.float32),
    )(xa, t, embed_table, w1a, w1b, b1)


@functools.partial(
    pl.kernel,
    mesh=_SC_MESH,
    out_type=jax.ShapeDtypeStruct((NCH, CHUNK, HIDDEN), jnp.float32),
    scratch_types=[
        pltpu.VMEM((CPW, CHUNK), jnp.int32),
        pltpu.VMEM((CHUNK, HIDDEN), jnp.float32),
        pltpu.SemaphoreType.DMA,
    ],
    compiler_params=pltpu.CompilerParams(use_tc_tiling_on_sc=False),
)
def _sc_gather(u_hbm, src_hbm, g_hbm, idx_v, rows_v, sem):
    wid = lax.axis_index("s") * 2 + lax.axis_index("c")
    base = wid * CPW
    pltpu.sync_copy(src_hbm.at[pl.ds(base, CPW)], idx_v)

    def body(c, carry):
        pltpu.async_copy(u_hbm.at[idx_v.at[c]], rows_v, sem).wait()
        pltpu.sync_copy(rows_v, g_hbm.at[base + c])
        return carry

    lax.fori_loop(0, CPW, body, 0)


def _mlp_body(g_ref, ea_ref, w1e_ref, wh_ref, bh_ref, h2_ref):
    h1 = jnp.maximum(
        g_ref[...] + jnp.dot(ea_ref[...], w1e_ref[...],
                             preferred_element_type=jnp.float32), 0.0)
    h2_ref[...] = jnp.maximum(
        jnp.dot(h1, wh_ref[...], preferred_element_type=jnp.float32)
        + bh_ref[...], 0.0)


def _edge_mlp(g, ea, w1e, wh, bh):
    be = 4096
    grid = EP // be
    return pl.pallas_call(
        _mlp_body,
        grid=(grid,),
        in_specs=[
            pl.BlockSpec((be, HIDDEN), lambda i: (i, 0)),
            pl.BlockSpec((be, EDGE_DIM), lambda i: (i, 0)),
            pl.BlockSpec((EDGE_DIM, HIDDEN), lambda i: (0, 0)),
            pl.BlockSpec((HIDDEN, HIDDEN), lambda i: (0, 0)),
            pl.BlockSpec((1, HIDDEN), lambda i: (0, 0)),
        ],
        out_specs=pl.BlockSpec((be, HIDDEN), lambda i: (i, 0)),
        out_shape=jax.ShapeDtypeStruct((EP, HIDDEN), jnp.float32),
    )(g, ea, w1e, wh, bh)


@functools.partial(
    pl.kernel,
    mesh=_SC_MESH,
    out_type=jax.ShapeDtypeStruct((2, ACC_ROWS, HIDDEN), jnp.float32),
    scratch_types=[
        pltpu.VMEM((CPW, CHUNK), jnp.int32),
        pltpu.VMEM((CHUNK, HIDDEN), jnp.float32),
        pltpu.VMEM((STRIPE, HIDDEN), jnp.float32),
        pltpu.VMEM_SHARED((ACC_ROWS, HIDDEN), jnp.float32),
    ],
    compiler_params=pltpu.CompilerParams(use_tc_tiling_on_sc=False),
)
def _sc_scatter(h2_hbm, dst_hbm, part_hbm, idx_v, rows_v, stripe_v, acc):
    cid = lax.axis_index("c")
    sid = lax.axis_index("s")
    wid = sid * 2 + cid
    base = wid * CPW

    zeros = jnp.zeros((16,), jnp.float32)

    def zbody(r, carry):
        for j in range(HIDDEN // 16):
            stripe_v[r, pl.ds(j * 16, 16)] = zeros
        return carry

    lax.fori_loop(0, STRIPE, zbody, 0)
    pltpu.sync_copy(stripe_v, acc.at[pl.ds(sid * STRIPE, STRIPE)])
    plsc.subcore_barrier()

    pltpu.sync_copy(dst_hbm.at[pl.ds(base, CPW)], idx_v)

    def body(c, carry):
        pltpu.sync_copy(h2_hbm.at[base + c], rows_v)
        pltpu.sync_copy(rows_v, acc.at[idx_v.at[c]], add=True)
        return carry

    lax.fori_loop(0, CPW, body, 0)
    plsc.subcore_barrier()

    pltpu.sync_copy(acc.at[pl.ds(sid * STRIPE, STRIPE)], stripe_v)
    pltpu.sync_copy(stripe_v, part_hbm.at[cid, pl.ds(sid * STRIPE, STRIPE)])


def _sum_body(a_ref, b_ref, o_ref):
    o_ref[...] = a_ref[...] + b_ref[...]


def _sum_partials(p0, p1):
    bn = 1000
    grid = N_NODES // bn
    return pl.pallas_call(
        _sum_body,
        grid=(grid,),
        in_specs=[
            pl.BlockSpec((bn, HIDDEN), lambda i: (i, 0)),
            pl.BlockSpec((bn, HIDDEN), lambda i: (i, 0)),
        ],
        out_specs=pl.BlockSpec((bn, HIDDEN), lambda i: (i, 0)),
        out_shape=jax.ShapeDtypeStruct((N_NODES, HIDDEN), jnp.float32),
    )(p0, p1)


@jax.jit
def kernel(x, edge_index, edge_attr, embed_table, W1, b1, Wh, bh):
    xa = x[:, :D_FEAT]
    t = x[:, D_FEAT:]
    w1a = W1[:D_FEAT]
    w1b = W1[D_FEAT:D_FEAT + EMB]
    w1e = W1[D_FEAT + EMB:]

    n_edges = edge_index.shape[1]
    pad = EP - n_edges
    src = jnp.concatenate(
        [edge_index[0], jnp.zeros((pad,), jnp.int32)]).reshape(NCH, CHUNK)
    dst = jnp.concatenate(
        [edge_index[1],
         jnp.full((pad,), N_NODES, jnp.int32)]).reshape(NCH, CHUNK)
    ea = jnp.concatenate(
        [edge_attr, jnp.zeros((pad, EDGE_DIM), jnp.float32)])

    u = _node_precompute(xa, t, embed_table, w1a, w1b,
                         b1.reshape(1, HIDDEN))
    g = _sc_gather(u, src)
    h2 = _edge_mlp(g.reshape(EP, HIDDEN), ea, w1e, Wh,
                   bh.reshape(1, HIDDEN))
    part = _sc_scatter(h2.reshape(NCH, CHUNK, HIDDEN), dst)
    return _sum_partials(part[0, :N_NODES], part[1, :N_NODES])
